# baseline (speedup 1.0000x reference)
import jax
import jax.numpy as jnp
from jax import lax
from jax.experimental import pallas as pl
from jax.experimental.pallas import tpu as pltpu
from jax.experimental.pallas import tpu_sc as plsc

N_ROWS = 1024
VOCAB = 100000
NEG = 64
TOT = N_ROWS * NEG
NC, NS = 2, 16
NW = NC * NS
CHUNK = TOT // NW
IDXW = 512
PIDXW = 256
NCH = CHUNK // IDXW
RPW = 256
L = 16

_LP = (1.69366266e-06, 0.999832595, -0.497203331, 0.31504128,
       -0.189019548, 0.0815231776, -0.0170296106)
_LOG2E = 1.4426950408889634


def _softplus16(x):
    e = jnp.exp(-jnp.abs(x))
    p = _LP[6]
    for c in (_LP[5], _LP[4], _LP[3], _LP[2], _LP[1], _LP[0]):
        p = p * e + c
    return jnp.maximum(x, 0.0) + p


def _sc_loss_partials(flat, nact_phys, target):
    def body(flat_hbm, nact_hbm, tgt_hbm, out_hbm,
             nact_v, tgt_v, pidx_v, nidx_v, pval_v, nval_v, acc_v,
             sem_t, sem_n, sem_p, sem_gs):
        wid = lax.axis_index("s") * NC + lax.axis_index("c")
        base = wid * CHUNK
        rowbase = (wid % 4) * RPW
        rw2 = (wid % 4) * 2
        cp_n = pltpu.async_copy(nact_hbm.at[pl.ds(base, CHUNK)], nact_v, sem_n)
        cp_t = pltpu.async_copy(tgt_hbm.at[pl.ds(rowbase, RPW)], tgt_v, sem_t)

        lanes = lax.iota(jnp.int32, L)

        cp_t.wait()
        for g in range(RPW // L):
            j = tgt_v[pl.ds(g * L, L)]
            vbase = (rw2 + (g >> 3)) * 1024 + (g & 7) * L + lanes
            pidx_v[pl.ds(g * L, L)] = ((j >> 3) << 13) | ((j & 7) << 7) | vbase
        pos_copies = [
            pltpu.async_copy(flat_hbm.at[pidx_v.at[pl.ds(k * PIDXW, PIDXW)]],
                             pval_v.at[pl.ds(k * PIDXW, PIDXW)], sem_p)
            for k in range(RPW // PIDXW)]

        cp_n.wait()
        neg_copies = []
        for k in range(NCH):
            def nidx_body(g, carry, _k=k):
                j = nact_v[pl.ds(g * L, L)]
                vbase = (rw2 + (g >> 6)) * 1024 + (g & 7) * L + lanes
                nidx_v[pl.ds(g * L, L)] = ((j >> 3) << 13) | ((j & 7) << 7) | vbase
                return carry

            lax.fori_loop(k * (IDXW // L), (k + 1) * (IDXW // L), nidx_body, 0,
                          unroll=4)
            neg_copies.append(
                pltpu.async_copy(flat_hbm.at[nidx_v.at[pl.ds(k * IDXW, IDXW)]],
                                 nval_v.at[pl.ds(k * IDXW, IDXW)], sem_gs.at[k]))
        for c in pos_copies:
            c.wait()
        for c in neg_copies:
            c.wait()

        zero = jnp.zeros((L,), jnp.float32)

        def loss_body(t, accs):
            res = list(accs)
            ptile = (t >> 3) * 128
            for u in range(8):
                neg = nval_v[pl.ds(t * 128 + u * L, L)]
                pos = pval_v[pl.ds(ptile + u * L, L)]
                res[u] = res[u] + _softplus16(neg - pos)
            return tuple(res)

        accs = lax.fori_loop(0, CHUNK // 128, loss_body, (zero,) * 8)
        acc_v[...] = ((accs[0] + accs[1]) + (accs[2] + accs[3])) + (
            (accs[4] + accs[5]) + (accs[6] + accs[7]))
        pltpu.sync_copy(acc_v, out_hbm.at[wid])

    fn = pl.kernel(
        body,
        out_type=jax.ShapeDtypeStruct((NW, L), jnp.float32),
        mesh=plsc.VectorSubcoreMesh(core_axis_name="c", subcore_axis_name="s"),
        scratch_types=[
            pltpu.VMEM((CHUNK,), jnp.int32),
            pltpu.VMEM((RPW,), jnp.int32),
            pltpu.VMEM((RPW,), jnp.int32),
            pltpu.VMEM((CHUNK,), jnp.int32),
            pltpu.VMEM((RPW,), jnp.float32),
            pltpu.VMEM((CHUNK,), jnp.float32),
            pltpu.VMEM((L,), jnp.float32),
            pltpu.SemaphoreType.DMA,
            pltpu.SemaphoreType.DMA,
            pltpu.SemaphoreType.DMA,
            pltpu.SemaphoreType.DMA((NCH,)),
        ],
    )
    return fn(flat, nact_phys, target)


def _tc_reduce(partials):
    def body(p_ref, out_ref):
        out_ref[0, 0] = jnp.sum(p_ref[...]) * (1.0 / TOT)

    return pl.pallas_call(
        body,
        out_shape=jax.ShapeDtypeStruct((1, 1), jnp.float32),
        out_specs=pl.BlockSpec(memory_space=pltpu.SMEM),
    )(partials)


def kernel(input, target, neg_action):
    flat = input.reshape(8, 128, VOCAB // 8, 8).transpose(2, 0, 3, 1).reshape(-1)
    nact_phys = neg_action.reshape(8, 128, NEG // 8, 8).transpose(2, 0, 3, 1).reshape(-1)
    partials = _sc_loss_partials(flat, nact_phys, target)
    return _tc_reduce(partials)[0, 0]


# speedup vs baseline: 1.0100x; 1.0100x over previous
import jax
import jax.numpy as jnp
from jax import lax
from jax.experimental import pallas as pl
from jax.experimental.pallas import tpu as pltpu
from jax.experimental.pallas import tpu_sc as plsc

N_ROWS = 1024
VOCAB = 100000
NEG = 64
TOT = N_ROWS * NEG
NC, NS = 2, 16
NW = NC * NS
CHUNK = TOT // NW
IDXW = 512
PIDXW = 256
NCH = CHUNK // IDXW
RPW = 256
L = 16

_LP = (1.69366266e-06, 0.999832595, -0.497203331, 0.31504128,
       -0.189019548, 0.0815231776, -0.0170296106)
_LOG2E = 1.4426950408889634


def _softplus16(x):
    e = jnp.exp(-jnp.abs(x))
    p = _LP[6]
    for c in (_LP[5], _LP[4], _LP[3], _LP[2], _LP[1], _LP[0]):
        p = p * e + c
    return jnp.maximum(x, 0.0) + p


def _sc_loss_partials(flat, nact_phys, target):
    def body(flat_hbm, nact_hbm, tgt_hbm, out_hbm,
             nact_v, tgt_v, pidx_v, nidx_v, pval_v, nval_v, acc_v,
             sem_t, sem_n, sem_p, sem_gs):
        wid = lax.axis_index("s") * NC + lax.axis_index("c")
        base = wid * CHUNK
        rowbase = (wid % 4) * RPW
        rw2 = (wid % 4) * 2
        cp_n = pltpu.async_copy(nact_hbm.at[pl.ds(base, CHUNK)], nact_v, sem_n)
        cp_t = pltpu.async_copy(tgt_hbm.at[pl.ds(rowbase, RPW)], tgt_v, sem_t)

        lanes = lax.iota(jnp.int32, L)

        cp_t.wait()
        for g in range(RPW // L):
            j = tgt_v[pl.ds(g * L, L)]
            vbase = (rw2 + (g >> 3)) * 1024 + (g & 7) * L + lanes
            pidx_v[pl.ds(g * L, L)] = ((j >> 3) << 13) | ((j & 7) << 7) | vbase
        pos_copies = [
            pltpu.async_copy(flat_hbm.at[pidx_v.at[pl.ds(k * PIDXW, PIDXW)]],
                             pval_v.at[pl.ds(k * PIDXW, PIDXW)], sem_p)
            for k in range(RPW // PIDXW)]

        cp_n.wait()
        neg_copies = []
        for k in range(NCH):
            for u in range(IDXW // L):
                g = k * (IDXW // L) + u
                j = nact_v[pl.ds(g * L, L)]
                vbase = (rw2 + (g >> 6)) * 1024 + (g & 7) * L + lanes
                nidx_v[pl.ds(g * L, L)] = ((j >> 3) << 13) | ((j & 7) << 7) | vbase
            neg_copies.append(
                pltpu.async_copy(flat_hbm.at[nidx_v.at[pl.ds(k * IDXW, IDXW)]],
                                 nval_v.at[pl.ds(k * IDXW, IDXW)], sem_gs.at[k]))
        for c in pos_copies:
            c.wait()
        for c in neg_copies:
            c.wait()

        zero = jnp.zeros((L,), jnp.float32)

        def loss_body(t, accs):
            res = list(accs)
            ptile = (t >> 3) * 128
            for u in range(8):
                neg = nval_v[pl.ds(t * 128 + u * L, L)]
                pos = pval_v[pl.ds(ptile + u * L, L)]
                res[u] = res[u] + _softplus16(neg - pos)
            return tuple(res)

        accs = lax.fori_loop(0, CHUNK // 128, loss_body, (zero,) * 8)
        acc_v[...] = ((accs[0] + accs[1]) + (accs[2] + accs[3])) + (
            (accs[4] + accs[5]) + (accs[6] + accs[7]))
        pltpu.sync_copy(acc_v, out_hbm.at[wid])

    fn = pl.kernel(
        body,
        out_type=jax.ShapeDtypeStruct((NW, L), jnp.float32),
        mesh=plsc.VectorSubcoreMesh(core_axis_name="c", subcore_axis_name="s"),
        scratch_types=[
            pltpu.VMEM((CHUNK,), jnp.int32),
            pltpu.VMEM((RPW,), jnp.int32),
            pltpu.VMEM((RPW,), jnp.int32),
            pltpu.VMEM((CHUNK,), jnp.int32),
            pltpu.VMEM((RPW,), jnp.float32),
            pltpu.VMEM((CHUNK,), jnp.float32),
            pltpu.VMEM((L,), jnp.float32),
            pltpu.SemaphoreType.DMA,
            pltpu.SemaphoreType.DMA,
            pltpu.SemaphoreType.DMA,
            pltpu.SemaphoreType.DMA((NCH,)),
        ],
    )
    return fn(flat, nact_phys, target)


def _tc_reduce(partials):
    def body(p_ref, out_ref):
        out_ref[0, 0] = jnp.sum(p_ref[...]) * (1.0 / TOT)

    return pl.pallas_call(
        body,
        out_shape=jax.ShapeDtypeStruct((1, 1), jnp.float32),
        out_specs=pl.BlockSpec(memory_space=pltpu.SMEM),
    )(partials)


def kernel(input, target, neg_action):
    flat = input.reshape(8, 128, VOCAB // 8, 8).transpose(2, 0, 3, 1).reshape(-1)
    nact_phys = neg_action.reshape(8, 128, NEG // 8, 8).transpose(2, 0, 3, 1).reshape(-1)
    partials = _sc_loss_partials(flat, nact_phys, target)
    return _tc_reduce(partials)[0, 0]
